# Initial kernel scaffold; baseline (speedup 1.0000x reference)
import functools

import jax
import jax.numpy as jnp
from jax import lax
from jax.experimental import pallas as pl
from jax.experimental.pallas import tpu as pltpu
from jax.experimental.pallas import tpu_sc as plsc

NC = 2
NS = 16
NW = NC * NS

BATCH = 64
SEQ = 8192
DIM = 64
CHUNK = SEQ // NW


def _sc_body(x_hbm, tab_hbm, pe_hbm, out_hbm, idx_v, pe_v, buf, sem):
    wid = lax.axis_index("s") * NC + lax.axis_index("c")
    l0 = wid * CHUNK

    pltpu.sync_copy(pe_hbm.at[pl.ds(l0, CHUNK)], pe_v)

    def batch_body(b, carry):
        pltpu.sync_copy(x_hbm.at[b, pl.ds(l0, CHUNK)], idx_v)
        pltpu.async_copy(tab_hbm.at[idx_v], buf, sem).wait()

        def row_body(r, c2):
            for c in range(DIM // 16):
                sl = pl.ds(c * 16, 16)
                buf[r, sl] = buf[r, sl] + pe_v[r, sl]
            return c2

        lax.fori_loop(0, CHUNK, row_body, 0)
        pltpu.sync_copy(buf, out_hbm.at[b, pl.ds(l0, CHUNK)])
        return carry

    lax.fori_loop(0, BATCH, batch_body, 0)


@jax.jit
def kernel(x, embedding_table, positional_encoding):
    pe2d = positional_encoding.reshape(SEQ, DIM)
    xi = x.astype(jnp.int32)

    mesh = plsc.VectorSubcoreMesh(
        core_axis_name="c", subcore_axis_name="s", num_cores=NC, num_subcores=NS
    )
    run = pl.kernel(
        _sc_body,
        out_type=jax.ShapeDtypeStruct((BATCH, SEQ, DIM), jnp.float32),
        mesh=mesh,
        scratch_types=[
            pltpu.VMEM((CHUNK,), jnp.int32),
            pltpu.VMEM((CHUNK, DIM), jnp.float32),
            pltpu.VMEM((CHUNK, DIM), jnp.float32),
            pltpu.SemaphoreType.DMA,
        ],
    )
    return run(xi, embedding_table, pe2d)


# speedup vs baseline: 3.7704x; 3.7704x over previous
import functools

import jax
import jax.numpy as jnp
from jax import lax
from jax.experimental import pallas as pl
from jax.experimental.pallas import tpu as pltpu
from jax.experimental.pallas import tpu_sc as plsc

NC = 2
NS = 16
NW = NC * NS

BATCH = 64
SEQ = 8192
DIM = 64
CHUNK = SEQ // NW


def _sc_body(x_hbm, tab_hbm, pe_hbm, out_hbm, idx_v, pe_v, buf, sem):
    wid = lax.axis_index("s") * NC + lax.axis_index("c")
    l0 = wid * CHUNK

    pltpu.sync_copy(pe_hbm.at[pl.ds(l0, CHUNK)], pe_v)

    def batch_body(b, carry):
        pltpu.sync_copy(x_hbm.at[b, pl.ds(l0, CHUNK)], idx_v)
        pltpu.async_copy(tab_hbm.at[idx_v], buf, sem).wait()

        def row_body(r, c2):
            for c in range(DIM // 16):
                sl = pl.ds(c * 16, 16)
                buf[r, sl] = buf[r, sl] + pe_v[r, sl]
            return c2

        lax.fori_loop(0, CHUNK, row_body, 0)
        pltpu.sync_copy(buf, out_hbm.at[b, pl.ds(l0, CHUNK)])
        return carry

    lax.fori_loop(0, BATCH, batch_body, 0)


@jax.jit
def kernel(x, embedding_table, positional_encoding):
    pe2d = positional_encoding.reshape(SEQ, DIM)
    xi = x.astype(jnp.int32)

    mesh = plsc.VectorSubcoreMesh(
        core_axis_name="c", subcore_axis_name="s", num_cores=NC, num_subcores=NS
    )
    run = pl.kernel(
        _sc_body,
        out_type=jax.ShapeDtypeStruct((BATCH, SEQ, DIM), jnp.float32),
        mesh=mesh,
        scratch_types=[
            pltpu.VMEM((CHUNK,), jnp.int32),
            pltpu.VMEM((CHUNK, DIM), jnp.float32),
            pltpu.VMEM((CHUNK, DIM), jnp.float32),
            pltpu.SemaphoreType.DMA,
        ],
        compiler_params=pltpu.CompilerParams(use_tc_tiling_on_sc=False),
    )
    return run(xi, embedding_table, pe2d)


# speedup vs baseline: 4.7041x; 1.2476x over previous
import jax
import jax.numpy as jnp
from jax import lax
from jax.experimental import pallas as pl
from jax.experimental.pallas import tpu as pltpu
from jax.experimental.pallas import tpu_sc as plsc

NC = 2
NS = 16
NW = NC * NS

BATCH = 64
SEQ = 8192
DIM = 64
CHUNK = SEQ // NW


def _sc_body(x_hbm, tab_hbm, pe_hbm, out_hbm,
             idx_all, pe_v, buf0, buf1, gsem0, gsem1, ssem0, ssem1):
    wid = lax.axis_index("s") * NC + lax.axis_index("c")
    l0 = wid * CHUNK

    pltpu.sync_copy(pe_hbm.at[pl.ds(l0, CHUNK)], pe_v)
    pltpu.sync_copy(x_hbm.at[:, pl.ds(l0, CHUNK)], idx_all)

    bufs = (buf0, buf1)
    gsems = (gsem0, gsem1)
    ssems = (ssem0, ssem1)

    for p in range(2):
        pltpu.async_copy(tab_hbm.at[idx_all.at[p]], bufs[p], gsems[p])

    def step(b2, carry):
        for p in range(2):
            b = b2 * 2 + p
            buf = bufs[p]
            pltpu.make_async_copy(tab_hbm.at[idx_all.at[b]], buf, gsems[p]).wait()

            def row_body(r, c2):
                for c in range(DIM // 16):
                    sl = pl.ds(c * 16, 16)
                    buf[r, sl] = buf[r, sl] + pe_v[r, sl]
                return c2

            lax.fori_loop(0, CHUNK, row_body, 0)
            pltpu.async_copy(buf, out_hbm.at[b, pl.ds(l0, CHUNK)], ssems[p])

        @pl.when(b2 < BATCH // 2 - 1)
        def _():
            for p in range(2):
                b = b2 * 2 + p
                pltpu.make_async_copy(
                    bufs[p], out_hbm.at[b, pl.ds(l0, CHUNK)], ssems[p]
                ).wait()
                pltpu.async_copy(
                    tab_hbm.at[idx_all.at[b + 2]], bufs[p], gsems[p]
                )

        return carry

    lax.fori_loop(0, BATCH // 2, step, 0)

    for p in range(2):
        b = BATCH - 2 + p
        pltpu.make_async_copy(
            bufs[p], out_hbm.at[b, pl.ds(l0, CHUNK)], ssems[p]
        ).wait()


@jax.jit
def kernel(x, embedding_table, positional_encoding):
    pe2d = positional_encoding.reshape(SEQ, DIM)
    xi = x.astype(jnp.int32)

    mesh = plsc.VectorSubcoreMesh(
        core_axis_name="c", subcore_axis_name="s", num_cores=NC, num_subcores=NS
    )
    run = pl.kernel(
        _sc_body,
        out_type=jax.ShapeDtypeStruct((BATCH, SEQ, DIM), jnp.float32),
        mesh=mesh,
        scratch_types=[
            pltpu.VMEM((BATCH, CHUNK), jnp.int32),
            pltpu.VMEM((CHUNK, DIM), jnp.float32),
            pltpu.VMEM((CHUNK, DIM), jnp.float32),
            pltpu.VMEM((CHUNK, DIM), jnp.float32),
            pltpu.SemaphoreType.DMA,
            pltpu.SemaphoreType.DMA,
            pltpu.SemaphoreType.DMA,
            pltpu.SemaphoreType.DMA,
        ],
        compiler_params=pltpu.CompilerParams(use_tc_tiling_on_sc=False),
    )
    return run(xi, embedding_table, pe2d)


# speedup vs baseline: 5.0802x; 1.0800x over previous
import jax
import jax.numpy as jnp
from jax import lax
from jax.experimental import pallas as pl
from jax.experimental.pallas import tpu as pltpu
from jax.experimental.pallas import tpu_sc as plsc

NC = 2
NS = 16
NW = NC * NS

BATCH = 64
SEQ = 8192
DIM = 64
PAD = 128
CHUNK = SEQ // NW
HALF = CHUNK // 2


def _sc_body(x_hbm, tab_hbm, pe_hbm, out_hbm,
             idx0, idx1, pe_v, buf0, buf1, sbuf0, sbuf1,
             gsem0, gsem1, ssem0, ssem1, isem0, isem1):
    wid = lax.axis_index("s") * NC + lax.axis_index("c")
    l0 = wid * CHUNK

    pltpu.sync_copy(pe_hbm.at[pl.ds(l0, CHUNK)], pe_v)

    idxs = (idx0, idx1)
    bufs = (buf0, buf1)
    sbufs = (sbuf0, sbuf1)
    gsems = (gsem0, gsem1)
    ssems = (ssem0, ssem1)
    isems = (isem0, isem1)

    for p in range(2):
        pltpu.sync_copy(x_hbm.at[0, pl.ds(l0 + p * HALF, HALF)], idxs[p])
        pltpu.async_copy(tab_hbm.at[idxs[p]], bufs[p], gsems[p])

    def step(b, carry):
        for p in range(2):
            off = p * HALF
            buf, sbuf = bufs[p], sbufs[p]
            pltpu.make_async_copy(tab_hbm.at[idxs[p]], buf, gsems[p]).wait()

            @pl.when(b < BATCH - 1)
            def _():
                pltpu.async_copy(
                    x_hbm.at[b + 1, pl.ds(l0 + off, HALF)], idxs[p], isems[p]
                )

            def row_body(r, c2):
                for c in range(DIM // 16):
                    sl = pl.ds(c * 16, 16)
                    sbuf[r, sl] = buf[r, sl] + pe_v[r + off, sl]
                return c2

            lax.fori_loop(0, HALF, row_body, 0)
            pltpu.async_copy(sbuf, out_hbm.at[b, pl.ds(l0 + off, HALF)], ssems[p])

        @pl.when(b < BATCH - 1)
        def _():
            for p in range(2):
                off = p * HALF
                pltpu.make_async_copy(
                    sbufs[p], out_hbm.at[b, pl.ds(l0 + off, HALF)], ssems[p]
                ).wait()
                pltpu.make_async_copy(
                    x_hbm.at[b + 1, pl.ds(l0 + off, HALF)], idxs[p], isems[p]
                ).wait()
                pltpu.async_copy(tab_hbm.at[idxs[p]], bufs[p], gsems[p])

        return carry

    lax.fori_loop(0, BATCH, step, 0)

    for p in range(2):
        off = p * HALF
        pltpu.make_async_copy(
            sbufs[p], out_hbm.at[BATCH - 1, pl.ds(l0 + off, HALF)], ssems[p]
        ).wait()


@jax.jit
def kernel(x, embedding_table, positional_encoding):
    pe2d = positional_encoding.reshape(SEQ, DIM)
    xi = x.astype(jnp.int32)
    tab_pad = jnp.pad(embedding_table, ((0, 0), (0, PAD - DIM)))

    mesh = plsc.VectorSubcoreMesh(
        core_axis_name="c", subcore_axis_name="s", num_cores=NC, num_subcores=NS
    )
    run = pl.kernel(
        _sc_body,
        out_type=jax.ShapeDtypeStruct((BATCH, SEQ, DIM), jnp.float32),
        mesh=mesh,
        scratch_types=[
            pltpu.VMEM((HALF,), jnp.int32),
            pltpu.VMEM((HALF,), jnp.int32),
            pltpu.VMEM((CHUNK, DIM), jnp.float32),
            pltpu.VMEM((HALF, PAD), jnp.float32),
            pltpu.VMEM((HALF, PAD), jnp.float32),
            pltpu.VMEM((HALF, DIM), jnp.float32),
            pltpu.VMEM((HALF, DIM), jnp.float32),
            pltpu.SemaphoreType.DMA,
            pltpu.SemaphoreType.DMA,
            pltpu.SemaphoreType.DMA,
            pltpu.SemaphoreType.DMA,
            pltpu.SemaphoreType.DMA,
            pltpu.SemaphoreType.DMA,
        ],
    )
    return run(xi, tab_pad, pe2d)
